# Initial kernel scaffold; baseline (speedup 1.0000x reference)
import jax
import jax.numpy as jnp
from jax.experimental import pallas as pl


def kernel(x, Wr, br, W1, b1, W2, b2):
    raise NotImplementedError("write your pallas kernel here")


# speedup vs baseline: 6.2718x; 6.2718x over previous
import jax
import jax.numpy as jnp
from jax.experimental import pallas as pl
from jax.experimental.pallas import tpu as pltpu

N = 8192
D = 2048
E = 64
K = 8
CAP = float(int(1.25 * N / E))

EPAD = 128
TILE = 512
NBLK = N // TILE
NEG = -1e30


def _body(x_ref, wrt_ref, br_ref,
          out_ref, loss_ref, dist_ref, conf_ref,
          counts_ref, csum_ref):
    i = pl.program_id(0)

    out_ref[...] = jnp.zeros_like(out_ref)

    logits = jnp.dot(x_ref[...], wrt_ref[...],
                     preferred_element_type=jnp.float32) + br_ref[...]

    m = jnp.max(logits, axis=1, keepdims=True)
    conf_tile = jnp.sum(m)
    vals = logits
    for _ in range(K - 1):
        vals = jnp.where(vals >= m, jnp.float32(NEG), vals)
        m = jnp.max(vals, axis=1, keepdims=True)
    member = (logits >= m).astype(jnp.float32)
    counts_tile = jnp.sum(member, axis=0, keepdims=True)

    @pl.when(i == 0)
    def _():
        counts_ref[...] = jnp.zeros_like(counts_ref)
        csum_ref[...] = jnp.zeros_like(csum_ref)

    counts_ref[...] += counts_tile
    csum_ref[...] += conf_tile

    @pl.when(i == NBLK - 1)
    def _():
        load = jnp.minimum(counts_ref[...], jnp.float32(CAP))
        s = jnp.sum(load)
        dist = load / (s + jnp.float32(1e-8))
        dist_ref[...] = dist
        loss_ref[...] = jnp.sum(dist * jnp.log(dist + jnp.float32(1e-8))).reshape(1, 1)
        conf_ref[...] = csum_ref[...] * jnp.float32(1.0 / N)


def kernel(x, Wr, br, W1, b1, W2, b2):
    del W1, b1, W2, b2
    wrt = jnp.pad(Wr.T, ((0, 0), (0, EPAD - E)))
    brp = jnp.pad(br.reshape(1, E), ((0, 0), (0, EPAD - E)),
                  constant_values=NEG)

    out, loss, dist, conf = pl.pallas_call(
        _body,
        grid=(NBLK,),
        in_specs=[
            pl.BlockSpec((TILE, D), lambda i: (i, 0)),
            pl.BlockSpec((D, EPAD), lambda i: (0, 0)),
            pl.BlockSpec((1, EPAD), lambda i: (0, 0)),
        ],
        out_specs=[
            pl.BlockSpec((TILE, D), lambda i: (i, 0)),
            pl.BlockSpec((1, 1), lambda i: (0, 0)),
            pl.BlockSpec((1, EPAD), lambda i: (0, 0)),
            pl.BlockSpec((1, 1), lambda i: (0, 0)),
        ],
        out_shape=[
            jax.ShapeDtypeStruct((N, D), jnp.float32),
            jax.ShapeDtypeStruct((1, 1), jnp.float32),
            jax.ShapeDtypeStruct((1, EPAD), jnp.float32),
            jax.ShapeDtypeStruct((1, 1), jnp.float32),
        ],
        scratch_shapes=[
            pltpu.VMEM((1, EPAD), jnp.float32),
            pltpu.VMEM((1, 1), jnp.float32),
        ],
    )(x, wrt, brp)

    return (out,
            loss.reshape(()),
            dist[0, :E],
            conf.reshape(()))


# speedup vs baseline: 6.5635x; 1.0465x over previous
import jax
import jax.numpy as jnp
from jax.experimental import pallas as pl
from jax.experimental.pallas import tpu as pltpu

N = 8192
D = 2048
E = 64
K = 8
CAP = float(int(1.25 * N / E))

EPAD = 128
TILE = 1024
NBLK = N // TILE
NEG = -1e30


def _body(x_ref, wrt_ref, br_ref,
          out_ref, loss_ref, dist_ref, conf_ref,
          counts_ref, csum_ref):
    i = pl.program_id(0)

    out_ref[...] = jnp.zeros_like(out_ref)

    logits = jnp.dot(x_ref[...], wrt_ref[...],
                     preferred_element_type=jnp.float32) + br_ref[...]

    m = jnp.max(logits, axis=1, keepdims=True)
    conf_tile = jnp.sum(m)
    vals = logits
    for _ in range(K - 1):
        vals = jnp.where(vals >= m, jnp.float32(NEG), vals)
        m = jnp.max(vals, axis=1, keepdims=True)
    member = (logits >= m).astype(jnp.float32)
    counts_tile = jnp.sum(member, axis=0, keepdims=True)

    @pl.when(i == 0)
    def _():
        counts_ref[...] = jnp.zeros_like(counts_ref)
        csum_ref[...] = jnp.zeros_like(csum_ref)

    counts_ref[...] += counts_tile
    csum_ref[...] += conf_tile

    @pl.when(i == NBLK - 1)
    def _():
        load = jnp.minimum(counts_ref[...], jnp.float32(CAP))
        s = jnp.sum(load)
        dist = load / (s + jnp.float32(1e-8))
        dist_ref[...] = dist
        loss_ref[...] = jnp.sum(dist * jnp.log(dist + jnp.float32(1e-8))).reshape(1, 1)
        conf_ref[...] = csum_ref[...] * jnp.float32(1.0 / N)


def kernel(x, Wr, br, W1, b1, W2, b2):
    del W1, b1, W2, b2
    wrt = jnp.pad(Wr.T, ((0, 0), (0, EPAD - E)))
    brp = jnp.pad(br.reshape(1, E), ((0, 0), (0, EPAD - E)),
                  constant_values=NEG)

    out, loss, dist, conf = pl.pallas_call(
        _body,
        grid=(NBLK,),
        in_specs=[
            pl.BlockSpec((TILE, D), lambda i: (i, 0)),
            pl.BlockSpec((D, EPAD), lambda i: (0, 0)),
            pl.BlockSpec((1, EPAD), lambda i: (0, 0)),
        ],
        out_specs=[
            pl.BlockSpec((TILE, D), lambda i: (i, 0)),
            pl.BlockSpec((1, 1), lambda i: (0, 0)),
            pl.BlockSpec((1, EPAD), lambda i: (0, 0)),
            pl.BlockSpec((1, 1), lambda i: (0, 0)),
        ],
        out_shape=[
            jax.ShapeDtypeStruct((N, D), jnp.float32),
            jax.ShapeDtypeStruct((1, 1), jnp.float32),
            jax.ShapeDtypeStruct((1, EPAD), jnp.float32),
            jax.ShapeDtypeStruct((1, 1), jnp.float32),
        ],
        scratch_shapes=[
            pltpu.VMEM((1, EPAD), jnp.float32),
            pltpu.VMEM((1, 1), jnp.float32),
        ],
    )(x, wrt, brp)

    return (out,
            loss.reshape(()),
            dist[0, :E],
            conf.reshape(()))
